# baseline (speedup 1.0000x reference)
import functools

import jax
import jax.numpy as jnp
from jax import lax
from jax.experimental import pallas as pl
from jax.experimental.pallas import tpu as pltpu
from jax.experimental.pallas import tpu_sc as plsc

B = 4096
L = 200
E = 64
O = 64
VOCAB = 1000000
NC = 2
NS = 16
NW = NC * NS
RPW = B // NW
LP = 256


def _make_pool2():
    mesh = plsc.VectorSubcoreMesh(core_axis_name="c", subcore_axis_name="s")

    @functools.partial(
        pl.kernel,
        mesh=mesh,
        compiler_params=pltpu.CompilerParams(use_tc_tiling_on_sc=True),
        out_type=jax.ShapeDtypeStruct((B, E), jnp.float32),
        scratch_types=[
            pltpu.VMEM((RPW, LP), jnp.int32),
            pltpu.VMEM((RPW, LP), jnp.int32),
            pltpu.VMEM((L, 128), jnp.float32),
            pltpu.VMEM((RPW, E), jnp.float32),
            pltpu.SemaphoreType.DMA,
        ],
    )
    def pool2(x_hbm, tp_hbm, sum_hbm, idx_v, pidx_v, rows_v, acc_v, sem):
        wid = lax.axis_index("s") * NC + lax.axis_index("c")
        base = wid * RPW
        pltpu.sync_copy(x_hbm.at[pl.ds(base, RPW)], idx_v)

        def pb(k, carry):
            r = k // (LP // 16)
            c = (k % (LP // 16)) * 16
            pidx_v[r, pl.ds(c, 16)] = jax.lax.shift_right_logical(
                idx_v[r, pl.ds(c, 16)], 1)
            return carry
        lax.fori_loop(0, RPW * (LP // 16), pb, 0)

        def row_body(r, carry):
            d0 = pltpu.async_copy(
                tp_hbm.at[pidx_v.at[r, pl.ds(0, 128)]],
                rows_v.at[pl.ds(0, 128)], sem)
            d1 = pltpu.async_copy(
                tp_hbm.at[pidx_v.at[r, pl.ds(128, L - 128)]],
                rows_v.at[pl.ds(128, L - 128)], sem)
            d0.wait()
            d1.wait()

            def red_chunk(q, s, width):
                hv = idx_v[r, pl.ds(q * 16, 16)] & 1
                for k in range(width):
                    o = hv[k] * 64
                    j = q * 16 + k
                    s = (s[0] + rows_v[j, pl.ds(o, 16)],
                         s[1] + rows_v[j, pl.ds(o + 16, 16)],
                         s[2] + rows_v[j, pl.ds(o + 32, 16)],
                         s[3] + rows_v[j, pl.ds(o + 48, 16)])
                return s

            z = jnp.zeros((16,), jnp.float32)
            s0, s1, s2, s3 = lax.fori_loop(
                0, L // 16, lambda q, s: red_chunk(q, s, 16), (z, z, z, z))
            s0, s1, s2, s3 = red_chunk(L // 16, (s0, s1, s2, s3), L % 16)
            acc_v[r, pl.ds(0, 16)] = s0
            acc_v[r, pl.ds(16, 16)] = s1
            acc_v[r, pl.ds(32, 16)] = s2
            acc_v[r, pl.ds(48, 16)] = s3
            return carry

        lax.fori_loop(0, RPW, row_body, 0)
        pltpu.sync_copy(acc_v, sum_hbm.at[pl.ds(base, RPW)])

    return pool2


_pool2 = _make_pool2()


def kernel(x, table, W, b):
    xp = jnp.pad(x.astype(jnp.int32), ((0, 0), (0, LP - L)))
    tp = table.reshape(VOCAB // 2, 2 * E)
    sums = _pool2(xp, tp)
    h = jnp.maximum(sums * (1.0 / L) @ W + b, 0.0)
    return h


# speedup vs baseline: 1.2581x; 1.2581x over previous
import functools

import jax
import jax.numpy as jnp
from jax import lax
from jax.experimental import pallas as pl
from jax.experimental.pallas import tpu as pltpu
from jax.experimental.pallas import tpu_sc as plsc

B = 4096
L = 200
E = 64
O = 64
VOCAB = 1000000

NC = 2
NS = 16
NW = NC * NS
RPW = B // NW
C0 = 128
C1 = L - C0
LP = 256

CRW = 31248
CK = 168
CNB = CRW // CK
CTAIL = VOCAB - CRW * NW


def _make_compact():
    mesh = plsc.VectorSubcoreMesh(core_axis_name="c", subcore_axis_name="s")

    @functools.partial(
        pl.kernel,
        mesh=mesh,
        compiler_params=pltpu.CompilerParams(use_tc_tiling_on_sc=True),
        out_type=jax.ShapeDtypeStruct((VOCAB * E,), jnp.float32),
        scratch_types=[
            pltpu.VMEM((CK, E), jnp.float32),
            pltpu.VMEM((CK, E), jnp.float32),
            pltpu.VMEM((CK * E,), jnp.float32),
            pltpu.VMEM((CK * E,), jnp.float32),
            pltpu.SemaphoreType.DMA,
            pltpu.SemaphoreType.DMA,
            pltpu.SemaphoreType.DMA,
            pltpu.SemaphoreType.DMA,
        ],
    )
    def compact(fmt_hbm, lin_hbm, inA, inB, outA, outB,
                siA, siB, soA, soB):
        wid = lax.axis_index("s") * NC + lax.axis_index("c")
        base = wid * CRW

        def start_in(k, buf, sem):
            pltpu.async_copy(fmt_hbm.at[pl.ds(base + k * CK, CK), :], buf, sem)

        def wait_in(buf, sem):
            pltpu.make_async_copy(
                fmt_hbm.at[pl.ds(0, CK), :], buf, sem).wait()

        def start_out(k, out_v, sem):
            pltpu.async_copy(
                out_v, lin_hbm.at[pl.ds((base + k * CK) * E, CK * E)], sem)

        def wait_out(out_v, sem):
            pltpu.make_async_copy(
                lin_hbm.at[pl.ds(0, CK * E)], out_v, sem).wait()

        def compact_blk(in_v, out_v, nrows):
            def rb(r, carry):
                for i in range(4):
                    out_v[pl.ds(r * E + 16 * i, 16)] = in_v[r, pl.ds(16 * i, 16)]
                return carry
            lax.fori_loop(0, nrows, rb, 0)

        start_in(0, inA, siA)

        def pair_body(g, carry):
            k0 = 2 * g
            k1 = 2 * g + 1
            start_in(k1, inB, siB)
            wait_in(inA, siA)

            @pl.when(k0 >= 2)
            def _():
                wait_out(outA, soA)

            compact_blk(inA, outA, CK)
            start_out(k0, outA, soA)

            @pl.when(k0 + 2 < CNB)
            def _():
                start_in(k0 + 2, inA, siA)

            wait_in(inB, siB)

            @pl.when(k1 >= 2)
            def _():
                wait_out(outB, soB)

            compact_blk(inB, outB, CK)
            start_out(k1, outB, soB)

            @pl.when(k1 + 2 < CNB)
            def _():
                start_in(k1 + 2, inB, siB)

            return carry

        lax.fori_loop(0, CNB // 2, pair_body, 0)
        wait_out(outA, soA)
        wait_out(outB, soB)

        @pl.when(wid == 0)
        def _():
            r0 = CRW * NW
            pltpu.sync_copy(fmt_hbm.at[pl.ds(r0, CTAIL), :],
                            inA.at[pl.ds(0, CTAIL), :])
            compact_blk(inA, outA, CTAIL)
            pltpu.sync_copy(outA.at[pl.ds(0, CTAIL * E)],
                            lin_hbm.at[pl.ds(r0 * E, CTAIL * E)])

    return compact


_compact = _make_compact()


def _make_pool():
    mesh = plsc.VectorSubcoreMesh(core_axis_name="c", subcore_axis_name="s")

    @functools.partial(
        pl.kernel,
        mesh=mesh,
        compiler_params=pltpu.CompilerParams(use_tc_tiling_on_sc=False),
        out_type=jax.ShapeDtypeStruct((B, E), jnp.float32),
        scratch_types=[
            pltpu.VMEM((RPW, LP), jnp.int32),
            pltpu.VMEM((L, E), jnp.float32),
            pltpu.VMEM((L, E), jnp.float32),
            pltpu.VMEM((RPW, E), jnp.float32),
            pltpu.SemaphoreType.DMA,
            pltpu.SemaphoreType.DMA,
        ],
    )
    def pool(x_hbm, table_hbm, sum_hbm, idx_v, rows0_v, rows1_v, acc_v,
             sem0, sem1):
        wid = lax.axis_index("s") * NC + lax.axis_index("c")
        base = wid * RPW
        pltpu.sync_copy(x_hbm.at[pl.ds(base, RPW)], idx_v)

        def start(r, rows_v, sem):
            pltpu.async_copy(
                table_hbm.at[idx_v.at[r, pl.ds(0, C0)]],
                rows_v.at[pl.ds(0, C0)], sem)
            pltpu.async_copy(
                table_hbm.at[idx_v.at[r, pl.ds(C0, C1)]],
                rows_v.at[pl.ds(C0, C1)], sem)

        def wait(rows_v, sem):
            pltpu.make_async_copy(
                table_hbm.at[pl.ds(0, L)], rows_v, sem).wait()

        def reduce_into(rows_v, r):
            def red_body(j, s):
                for q in range(4):
                    jj = j * 4 + q
                    s = (s[0] + rows_v[jj, pl.ds(0, 16)],
                         s[1] + rows_v[jj, pl.ds(16, 16)],
                         s[2] + rows_v[jj, pl.ds(32, 16)],
                         s[3] + rows_v[jj, pl.ds(48, 16)])
                return s

            z = jnp.zeros((16,), jnp.float32)
            s0, s1, s2, s3 = lax.fori_loop(0, L // 4, red_body, (z, z, z, z))
            acc_v[r, pl.ds(0, 16)] = s0
            acc_v[r, pl.ds(16, 16)] = s1
            acc_v[r, pl.ds(32, 16)] = s2
            acc_v[r, pl.ds(48, 16)] = s3

        start(0, rows0_v, sem0)

        def pair_body(g, carry):
            start(2 * g + 1, rows1_v, sem1)
            wait(rows0_v, sem0)
            reduce_into(rows0_v, 2 * g)

            @pl.when(g < RPW // 2 - 1)
            def _():
                start(2 * g + 2, rows0_v, sem0)

            wait(rows1_v, sem1)
            reduce_into(rows1_v, 2 * g + 1)
            return carry

        lax.fori_loop(0, RPW // 2, pair_body, 0)
        pltpu.sync_copy(acc_v, sum_hbm.at[pl.ds(base, RPW)])

    return pool


_pool = _make_pool()

_BM = 512


def _dense_body(s_ref, w_ref, b_ref, o_ref):
    h = jnp.dot(s_ref[...], w_ref[...], preferred_element_type=jnp.float32)
    o_ref[...] = jnp.maximum(h * (1.0 / L) + b_ref[...], 0.0)


def _dense(sums, w, b2):
    return pl.pallas_call(
        _dense_body,
        grid=(B // _BM,),
        in_specs=[
            pl.BlockSpec((_BM, E), lambda i: (i, 0)),
            pl.BlockSpec((E, O), lambda i: (0, 0)),
            pl.BlockSpec((1, O), lambda i: (0, 0)),
        ],
        out_specs=pl.BlockSpec((_BM, O), lambda i: (i, 0)),
        out_shape=jax.ShapeDtypeStruct((B, O), jnp.float32),
    )(sums, w, b2)


def kernel(x, table, W, b):
    xp = jnp.pad(x.astype(jnp.int32), ((0, 0), (0, LP - L)))
    lin = _compact(table)
    sums = _pool(xp, lin.reshape(VOCAB, E))
    return _dense(sums, W, b.reshape(1, O))
